# baseline (speedup 1.0000x reference)
import functools

import jax
import jax.numpy as jnp
import numpy as np
from jax.experimental import pallas as pl
from jax.experimental.pallas import tpu as pltpu

_N = 32768
_N1 = 128
_N2 = 256
_K = 8


def _dft_constants():
    n1 = np.arange(_N1)
    n2 = np.arange(_N2)
    k2h = np.arange(_N1)
    f1 = np.exp(-2j * np.pi * np.outer(n1, n1) / _N1)
    tt = np.exp(-2j * np.pi * np.outer(n2, n1) / _N)
    f2 = np.exp(-2j * np.pi * np.outer(k2h, n2) / _N2)
    as32 = lambda a: np.ascontiguousarray(a, dtype=np.float32)
    return (as32(f1.real), as32(f1.imag), as32(tt.real), as32(tt.imag),
            as32(f2.real), as32(f2.imag))


_F1R, _F1I, _TTR, _TTI, _F2R, _F2I = _dft_constants()


def _fft_topk_body(v_ref, f1r_ref, f1i_ref, ttr_ref, tti_ref, f2r_ref,
                   f2i_ref, o_ref):
    hi = jax.lax.Precision.HIGHEST
    dot = functools.partial(jnp.dot, precision=hi,
                            preferred_element_type=jnp.float32)
    v = v_ref[...]
    br = dot(v, f1r_ref[...])
    bi = dot(v, f1i_ref[...])
    ttr = ttr_ref[...]
    tti = tti_ref[...]
    cr = br * ttr - bi * tti
    ci = br * tti + bi * ttr
    f2r = f2r_ref[...]
    f2i = f2i_ref[...]
    dr = dot(f2r, cr) - dot(f2i, ci)
    di = dot(f2r, ci) + dot(f2i, cr)
    mag = jnp.sqrt(dr * dr + di * di)
    row_i = jax.lax.broadcasted_iota(jnp.int32, (_N1, _N1), 0)
    col_i = jax.lax.broadcasted_iota(jnp.int32, (_N1, _N1), 1)
    lin = col_i + _N1 * row_i
    work = jnp.where(lin == 0, -1.0, mag)
    mags, res, ims = [], [], []
    for _ in range(_K):
        mx = jnp.max(work)
        idx = jnp.min(jnp.where(work == mx, lin, _N))
        hit = lin == idx
        mags.append(mx)
        res.append(jnp.sum(jnp.where(hit, dr, 0.0)))
        ims.append(jnp.sum(jnp.where(hit, di, 0.0)))
        work = jnp.where(hit, -1.0, work)
    i8 = jax.lax.broadcasted_iota(jnp.int32, (1, _K), 1)
    zero = jnp.zeros((1, _K), jnp.float32)
    magv = zero
    rev = zero
    imv = zero
    for j in range(_K):
        magv = magv + jnp.where(i8 == j, mags[j], 0.0)
        rev = rev + jnp.where(i8 == j, res[j], 0.0)
        imv = imv + jnp.where(i8 == j, ims[j], 0.0)
    phv = jnp.arctan2(imv, rev)
    o_ref[...] = jnp.concatenate([magv, phv], axis=1).reshape(1, 1, 2 * _K)


def _fft_topk_call(v, rows, interpret=False):
    const_spec = lambda shape: pl.BlockSpec(shape, lambda i: (0, 0))
    return pl.pallas_call(
        _fft_topk_body,
        grid=(rows,),
        in_specs=[
            pl.BlockSpec((_N2, _N1), lambda i: (i, 0)),
            const_spec((_N1, _N1)),
            const_spec((_N1, _N1)),
            const_spec((_N2, _N1)),
            const_spec((_N2, _N1)),
            const_spec((_N1, _N2)),
            const_spec((_N1, _N2)),
        ],
        out_specs=pl.BlockSpec((1, 1, 2 * _K), lambda i: (i, 0, 0)),
        out_shape=jax.ShapeDtypeStruct((rows, 1, 2 * _K), jnp.float32),
        compiler_params=pltpu.CompilerParams(
            dimension_semantics=("arbitrary",)),
        interpret=interpret,
    )(v, _F1R, _F1I, _TTR, _TTI, _F2R, _F2I)


def kernel(x):
    rows = x.shape[0]
    v = x.reshape(rows, _N1, _N2).transpose(0, 2, 1).reshape(rows * _N2, _N1)
    out = _fft_topk_call(v, rows)
    return out.reshape(rows, 2 * _K)


# speedup vs baseline: 1.3576x; 1.3576x over previous
import functools

import jax
import jax.numpy as jnp
import numpy as np
from jax.experimental import pallas as pl
from jax.experimental.pallas import tpu as pltpu

_N = 32768
_N1 = 128
_N2 = 256
_K = 8
_R = 8


def _dft_constants():
    n1 = np.arange(_N1)
    n2 = np.arange(_N2)
    k2h = np.arange(_N1)
    f1 = np.exp(-2j * np.pi * np.outer(n1, n1) / _N1)
    tt = np.exp(-2j * np.pi * np.outer(n2, n1) / _N)
    tt = np.tile(tt, (_R, 1))
    f2 = np.exp(-2j * np.pi * np.outer(k2h, n2) / _N2)
    as32 = lambda a: np.ascontiguousarray(a, dtype=np.float32)
    return (as32(f1.real), as32(f1.imag), as32(tt.real), as32(tt.imag),
            as32(f2.real), as32(f2.imag))


_F1R, _F1I, _TTR, _TTI, _F2R, _F2I = _dft_constants()


def _topk_features(dr, di):
    mag = jnp.sqrt(dr * dr + di * di)
    row_i = jax.lax.broadcasted_iota(jnp.int32, (_N1, _N1), 0)
    col_i = jax.lax.broadcasted_iota(jnp.int32, (_N1, _N1), 1)
    lin = col_i + _N1 * row_i
    work = jnp.where(lin == 0, -1.0, mag)
    i8 = jax.lax.broadcasted_iota(jnp.int32, (1, _K), 1)
    magv = jnp.zeros((1, _K), jnp.float32)
    rev = jnp.zeros((1, _K), jnp.float32)
    imv = jnp.zeros((1, _K), jnp.float32)
    for j in range(_K):
        mx = jnp.max(work, keepdims=True)
        idx = jnp.min(jnp.where(work == mx, lin, _N), keepdims=True)
        hit = lin == idx
        sel = i8 == j
        magv = magv + jnp.where(sel, mx, 0.0)
        rev = rev + jnp.where(sel, jnp.sum(jnp.where(hit, dr, 0.0),
                                           keepdims=True), 0.0)
        imv = imv + jnp.where(sel, jnp.sum(jnp.where(hit, di, 0.0),
                                           keepdims=True), 0.0)
        work = jnp.where(hit, -1.0, work)
    phv = jnp.arctan2(imv, rev)
    return jnp.concatenate([magv, phv], axis=1)


def _fft_topk_body(v_ref, f1r_ref, f1i_ref, ttr_ref, tti_ref, f2r_ref,
                   f2i_ref, o_ref):
    dot = functools.partial(jnp.dot, preferred_element_type=jnp.float32)
    v = v_ref[...]
    br = dot(v, f1r_ref[...])
    bi = dot(v, f1i_ref[...])
    ttr = ttr_ref[...]
    tti = tti_ref[...]
    cr = br * ttr - bi * tti
    ci = br * tti + bi * ttr
    f2r = f2r_ref[...]
    f2i = f2i_ref[...]
    feats = []
    for r in range(_R):
        crr = cr[r * _N2:(r + 1) * _N2]
        cir = ci[r * _N2:(r + 1) * _N2]
        dr = dot(f2r, crr) - dot(f2i, cir)
        di = dot(f2r, cir) + dot(f2i, crr)
        feats.append(_topk_features(dr, di))
    o_ref[...] = jnp.concatenate(feats, axis=0).reshape(_R, 1, 2 * _K)


def _fft_topk_call(v, rows, interpret=False):
    const_spec = lambda shape: pl.BlockSpec(shape, lambda i: (0, 0))
    return pl.pallas_call(
        _fft_topk_body,
        grid=(rows // _R,),
        in_specs=[
            pl.BlockSpec((_R * _N2, _N1), lambda i: (i, 0)),
            const_spec((_N1, _N1)),
            const_spec((_N1, _N1)),
            const_spec((_R * _N2, _N1)),
            const_spec((_R * _N2, _N1)),
            const_spec((_N1, _N2)),
            const_spec((_N1, _N2)),
        ],
        out_specs=pl.BlockSpec((_R, 1, 2 * _K), lambda i: (i, 0, 0)),
        out_shape=jax.ShapeDtypeStruct((rows, 1, 2 * _K), jnp.float32),
        compiler_params=pltpu.CompilerParams(
            dimension_semantics=("arbitrary",)),
        interpret=interpret,
    )(v, _F1R, _F1I, _TTR, _TTI, _F2R, _F2I)


def kernel(x):
    rows = x.shape[0]
    v = x.reshape(rows, _N1, _N2).transpose(0, 2, 1).reshape(rows * _N2, _N1)
    out = _fft_topk_call(v, rows)
    return out.reshape(rows, 2 * _K)


# speedup vs baseline: 1.7364x; 1.2790x over previous
import functools

import jax
import jax.numpy as jnp
import numpy as np
from jax.experimental import pallas as pl
from jax.experimental.pallas import tpu as pltpu

_N = 32768
_N1 = 128
_N2 = 256
_K = 8
_R = 8


def _dft_constants():
    n1 = np.arange(_N1)
    n2 = np.arange(_N2)
    k2h = np.arange(_N1)
    f1 = np.exp(-2j * np.pi * np.outer(n1, n1) / _N1)
    tt = np.exp(-2j * np.pi * np.outer(n2, n1) / _N)
    tt = np.tile(tt, (_R, 1))
    f2 = np.exp(-2j * np.pi * np.outer(k2h, n2) / _N2)
    as32 = lambda a: np.ascontiguousarray(a, dtype=np.float32)
    return (as32(f1.real), as32(f1.imag), as32(tt.real), as32(tt.imag),
            as32(f2.real), as32(f2.imag))


_F1R, _F1I, _TTR, _TTI, _F2R, _F2I = _dft_constants()


def _select_topk(dr, di):
    msq = dr * dr + di * di
    row_i = jax.lax.broadcasted_iota(jnp.int32, (_N1, _N1), 0)
    col_i = jax.lax.broadcasted_iota(jnp.int32, (_N1, _N1), 1)
    lin = col_i + _N1 * row_i
    key = jax.lax.bitcast_convert_type(msq, jnp.int32)
    key = jnp.bitwise_or(jnp.bitwise_and(key, -16384), lin)
    key = jnp.where(lin == 0, 0, key)
    crows = []
    for _ in range(_K):
        cmax = jnp.max(key, axis=0, keepdims=True)
        crows.append(cmax)
        key = jnp.where(key == cmax, 0, key)
    cand = jnp.concatenate(crows, axis=0)
    sub8 = jax.lax.broadcasted_iota(jnp.int32, (_K, _N1), 0)
    lane8 = jax.lax.broadcasted_iota(jnp.int32, (_K, _N1), 1)
    keysel = jnp.zeros((_K, _N1), jnp.int32)
    for j in range(_K):
        mx = jnp.max(cand, keepdims=True)
        keysel = jnp.where(sub8 == j, mx, keysel)
        cand = jnp.where(cand == mx, 0, cand)
    idx = jnp.bitwise_and(keysel, 16383)
    rowi = jax.lax.shift_right_logical(idx, 7)
    coli = jnp.bitwise_and(idx, 127)
    rhot = (lane8 == rowi).astype(jnp.float32)
    chot = (lane8 == coli).astype(jnp.float32)
    dotx = functools.partial(jnp.dot, precision=jax.lax.Precision.HIGHEST,
                             preferred_element_type=jnp.float32)
    re = jnp.sum(dotx(rhot, dr) * chot, axis=1, keepdims=True)
    im = jnp.sum(dotx(rhot, di) * chot, axis=1, keepdims=True)
    return re, im


def _fft_topk_body(v_ref, f1r_ref, f1i_ref, ttr_ref, tti_ref, f2r_ref,
                   f2i_ref, o_ref):
    dot = functools.partial(jnp.dot, preferred_element_type=jnp.float32)
    v = v_ref[...]
    br = dot(v, f1r_ref[...])
    bi = dot(v, f1i_ref[...])
    ttr = ttr_ref[...]
    tti = tti_ref[...]
    cr = br * ttr - bi * tti
    ci = br * tti + bi * ttr
    f2r = f2r_ref[...]
    f2i = f2i_ref[...]
    res, ims = [], []
    for r in range(_R):
        crr = cr[r * _N2:(r + 1) * _N2]
        cir = ci[r * _N2:(r + 1) * _N2]
        dr = dot(f2r, crr) - dot(f2i, cir)
        di = dot(f2r, cir) + dot(f2i, crr)
        re, im = _select_topk(dr, di)
        res.append(re)
        ims.append(im)
    rmat = jnp.concatenate(res, axis=1)
    imat = jnp.concatenate(ims, axis=1)
    rmat = rmat.T
    imat = imat.T
    mags = jnp.sqrt(rmat * rmat + imat * imat)
    phs = jnp.arctan2(imat, rmat)
    o_ref[...] = jnp.concatenate([mags, phs], axis=1)


def _fft_topk_call(v, rows, interpret=False):
    const_spec = lambda shape: pl.BlockSpec(shape, lambda i: (0, 0))
    return pl.pallas_call(
        _fft_topk_body,
        grid=(rows // _R,),
        in_specs=[
            pl.BlockSpec((_R * _N2, _N1), lambda i: (i, 0)),
            const_spec((_N1, _N1)),
            const_spec((_N1, _N1)),
            const_spec((_R * _N2, _N1)),
            const_spec((_R * _N2, _N1)),
            const_spec((_N1, _N2)),
            const_spec((_N1, _N2)),
        ],
        out_specs=pl.BlockSpec((_R, 2 * _K), lambda i: (i, 0)),
        out_shape=jax.ShapeDtypeStruct((rows, 2 * _K), jnp.float32),
        compiler_params=pltpu.CompilerParams(
            dimension_semantics=("arbitrary",)),
        interpret=interpret,
    )(v, _F1R, _F1I, _TTR, _TTI, _F2R, _F2I)


def kernel(x):
    rows = x.shape[0]
    v = x.reshape(rows, _N1, _N2).transpose(0, 2, 1).reshape(rows * _N2, _N1)
    return _fft_topk_call(v, rows)


# speedup vs baseline: 1.8138x; 1.0446x over previous
import functools

import jax
import jax.numpy as jnp
import numpy as np
from jax.experimental import pallas as pl
from jax.experimental.pallas import tpu as pltpu

_N = 32768
_N1 = 128
_N2 = 256
_K = 8
_R = 8


def _dft_constants():
    n1 = np.arange(_N1)
    n2 = np.arange(_N2)
    k2h = np.arange(_N1)
    f1 = np.exp(-2j * np.pi * np.outer(n1, n1) / _N1)
    tt = np.exp(-2j * np.pi * np.outer(n2, n1) / _N)
    tt = np.tile(tt, (_R, 1))
    f2 = np.exp(-2j * np.pi * np.outer(k2h, n2) / _N2)
    as32 = lambda a: np.ascontiguousarray(a, dtype=np.float32)
    return (as32(f1.real), as32(f1.imag), as32(tt.real), as32(tt.imag),
            as32(f2.real), as32(f2.imag))


_F1R, _F1I, _TTR, _TTI, _F2R, _F2I = _dft_constants()


def _bitonic_merge_desc(arr):
    n = len(arr)
    if n == 1:
        return arr
    half = n // 2
    top = [jnp.maximum(arr[i], arr[i + half]) for i in range(half)]
    bot = [jnp.minimum(arr[i], arr[i + half]) for i in range(half)]
    return _bitonic_merge_desc(top) + _bitonic_merge_desc(bot)


def _merge_desc(a, b):
    return _bitonic_merge_desc(a + b[::-1])


def _select_topk(dr, di):
    msq = dr * dr + di * di
    row_i = jax.lax.broadcasted_iota(jnp.int32, (_N1, _N1), 0)
    col_i = jax.lax.broadcasted_iota(jnp.int32, (_N1, _N1), 1)
    lin = col_i + _N1 * row_i
    key = jax.lax.bitcast_convert_type(msq, jnp.int32)
    key = jnp.bitwise_or(jnp.bitwise_and(key, -16384), lin)
    key = jnp.where(lin == 0, 0, key)
    w = [key[8 * j:8 * j + 8] for j in range(16)]
    runs = [_merge_desc([w[2 * i]], [w[2 * i + 1]]) for i in range(8)]
    runs = [_merge_desc(runs[2 * i], runs[2 * i + 1]) for i in range(4)]
    runs = [_merge_desc(runs[2 * i], runs[2 * i + 1]) for i in range(2)]
    a, b = runs
    s = _bitonic_merge_desc([jnp.maximum(a[i], b[7 - i]) for i in range(8)])
    sub8 = jax.lax.broadcasted_iota(jnp.int32, (_K, _N1), 0)
    lane8 = jax.lax.broadcasted_iota(jnp.int32, (_K, _N1), 1)
    keysel = jnp.zeros((_K, _N1), jnp.int32)
    for j in range(_K):
        mx = jnp.max(s[0], keepdims=True)
        keysel = jnp.where(sub8 == j, mx, keysel)
        hit = s[0] == mx
        for i in range(_K - 1):
            s[i] = jnp.where(hit, s[i + 1], s[i])
        s[_K - 1] = jnp.where(hit, 0, s[_K - 1])
    idx = jnp.bitwise_and(keysel, 16383)
    rowi = jax.lax.shift_right_logical(idx, 7)
    coli = jnp.bitwise_and(idx, 127)
    rhot = (lane8 == rowi).astype(jnp.float32)
    chot = (lane8 == coli).astype(jnp.float32)
    dotx = functools.partial(jnp.dot, precision=jax.lax.Precision.HIGHEST,
                             preferred_element_type=jnp.float32)
    re = jnp.sum(dotx(rhot, dr) * chot, axis=1, keepdims=True)
    im = jnp.sum(dotx(rhot, di) * chot, axis=1, keepdims=True)
    return re, im


def _fft_topk_body(v_ref, f1r_ref, f1i_ref, ttr_ref, tti_ref, f2r_ref,
                   f2i_ref, o_ref):
    dot = functools.partial(jnp.dot, preferred_element_type=jnp.float32)
    v = v_ref[...]
    br = dot(v, f1r_ref[...])
    bi = dot(v, f1i_ref[...])
    ttr = ttr_ref[...]
    tti = tti_ref[...]
    cr = br * ttr - bi * tti
    ci = br * tti + bi * ttr
    f2r = f2r_ref[...]
    f2i = f2i_ref[...]
    res, ims = [], []
    for r in range(_R):
        crr = cr[r * _N2:(r + 1) * _N2]
        cir = ci[r * _N2:(r + 1) * _N2]
        dr = dot(f2r, crr) - dot(f2i, cir)
        di = dot(f2r, cir) + dot(f2i, crr)
        re, im = _select_topk(dr, di)
        res.append(re)
        ims.append(im)
    rmat = jnp.concatenate(res, axis=1)
    imat = jnp.concatenate(ims, axis=1)
    rmat = rmat.T
    imat = imat.T
    mags = jnp.sqrt(rmat * rmat + imat * imat)
    phs = jnp.arctan2(imat, rmat)
    o_ref[...] = jnp.concatenate([mags, phs], axis=1)


def _fft_topk_call(v, rows, interpret=False):
    const_spec = lambda shape: pl.BlockSpec(shape, lambda i: (0, 0))
    return pl.pallas_call(
        _fft_topk_body,
        grid=(rows // _R,),
        in_specs=[
            pl.BlockSpec((_R * _N2, _N1), lambda i: (i, 0)),
            const_spec((_N1, _N1)),
            const_spec((_N1, _N1)),
            const_spec((_R * _N2, _N1)),
            const_spec((_R * _N2, _N1)),
            const_spec((_N1, _N2)),
            const_spec((_N1, _N2)),
        ],
        out_specs=pl.BlockSpec((_R, 2 * _K), lambda i: (i, 0)),
        out_shape=jax.ShapeDtypeStruct((rows, 2 * _K), jnp.float32),
        compiler_params=pltpu.CompilerParams(
            dimension_semantics=("arbitrary",)),
        interpret=interpret,
    )(v, _F1R, _F1I, _TTR, _TTI, _F2R, _F2I)


def kernel(x):
    rows = x.shape[0]
    v = x.reshape(rows, _N1, _N2).transpose(0, 2, 1).reshape(rows * _N2, _N1)
    return _fft_topk_call(v, rows)
